# Initial kernel scaffold; baseline (speedup 1.0000x reference)
import jax
import jax.numpy as jnp
from jax.experimental import pallas as pl


def kernel(xc, voxel, offset, scale, ratio, ratio_dim):
    raise NotImplementedError("write your pallas kernel here")


# speedup vs baseline: 1.1984x; 1.1984x over previous
import functools

import jax
import jax.numpy as jnp
from jax import lax
from jax.experimental import pallas as pl
from jax.experimental.pallas import tpu as pltpu
from jax.experimental.pallas import tpu_sc as plsc

L = 16


def _iota():
    return lax.broadcasted_iota(jnp.int32, (L,), 0)


@functools.cache
def _build_sc_call(M, C, D, H, W):
    info = plsc.get_sparse_core_info()
    NC, NS = info.num_cores, info.num_subcores
    NW = NC * NS
    PW = M // NW
    B = 64
    G = B // L
    NCHUNK = PW // B
    assert M % NW == 0 and PW % B == 0

    mesh = plsc.VectorSubcoreMesh(core_axis_name="c", subcore_axis_name="s")

    @functools.partial(
        pl.kernel,
        out_type=jax.ShapeDtypeStruct((M, C), jnp.float32),
        mesh=mesh,
        compiler_params=pltpu.CompilerParams(needs_layout_passes=False,
                                             use_tc_tiling_on_sc=False),
        scratch_types=[
            pltpu.VMEM((B * 3,), jnp.float32),
            pltpu.VMEM((G, 8 * L), jnp.int32),
            pltpu.VMEM((G * 8 * L, C), jnp.float32),
            pltpu.VMEM((G, 8, L), jnp.float32),
            pltpu.VMEM((B, C), jnp.float32),
            pltpu.VMEM((8, L), jnp.float32),
            pltpu.SemaphoreType.DMA,
        ],
    )
    def grid_sample_sc(xc_hbm, table_hbm, par_hbm, out_hbm,
                       coords_v, idx_v, rows_v, w_v, out_v, par_v, sem):
        wid = lax.axis_index("s") * NC + lax.axis_index("c")
        wbase = wid * PW
        pltpu.sync_copy(par_hbm, par_v)
        lane = _iota()

        def chunk_body(ci, carry):
            base = wbase + ci * B
            pltpu.sync_copy(xc_hbm.at[pl.ds(base * 3, B * 3)], coords_v)

            for g in range(G):
                a3 = (g * L + lane) * 3
                gx = plsc.load_gather(coords_v, [a3])
                gy = plsc.load_gather(coords_v, [a3 + 1])
                gz = plsc.load_gather(coords_v, [a3 + 2])

                def axis(gc, arow, brow, size):
                    A = par_v[arow]
                    Bc = par_v[brow]
                    i = jnp.minimum(jnp.maximum(A * gc + Bc, 0.0),
                                    float(size - 1))
                    i0 = i.astype(jnp.int32)
                    w1 = i - i0.astype(jnp.float32)
                    i1 = jnp.minimum(i0 + 1, size - 1)
                    return i0, i1, w1

                x0, x1, wx = axis(gx, 0, 1, W)
                y0, y1, wy = axis(gy, 2, 3, H)
                z0, z1, wz = axis(gz, 4, 5, D)

                ux0, ux1 = 1.0 - wx, wx
                uy0, uy1 = 1.0 - wy, wy
                uz0, uz1 = 1.0 - wz, wz
                zy = [z0 * (H * W) + y0 * W, z0 * (H * W) + y1 * W,
                      z1 * (H * W) + y0 * W, z1 * (H * W) + y1 * W]
                wzy = [uz0 * uy0, uz0 * uy1, uz1 * uy0, uz1 * uy1]
                for k in range(8):
                    dz_dy = (k >> 1)
                    dx = k & 1
                    r = zy[dz_dy] + (x0 if dx == 0 else x1)
                    wk = wzy[dz_dy] * (ux0 if dx == 0 else ux1)
                    idx_v[g, pl.ds(k * L, L)] = r
                    w_v[g, k] = wk

            descs = [
                pltpu.async_copy(table_hbm.at[idx_v.at[g]],
                                 rows_v.at[pl.ds(g * 8 * L, 8 * L)], sem)
                for g in range(G)
            ]
            for d in descs:
                d.wait()

            for g in range(G):
                rbase = [g * 8 * L + k * L + lane for k in range(8)]
                wk = [w_v[g, k] for k in range(8)]
                orow = g * L + lane
                for c in range(C):
                    col = jnp.full((L,), c, jnp.int32)
                    acc = plsc.load_gather(rows_v, [rbase[0], col]) * wk[0]
                    for k in range(1, 8):
                        acc = acc + plsc.load_gather(rows_v, [rbase[k], col]) * wk[k]
                    plsc.store_scatter(out_v, [orow, col], acc)

            pltpu.sync_copy(out_v, out_hbm.at[pl.ds(base, B)])
            return carry

        lax.fori_loop(0, NCHUNK, chunk_body, 0)

    return grid_sample_sc


def kernel(xc, voxel, offset, scale, ratio, ratio_dim):
    shape = xc.shape
    C, D, H, W = voxel.shape
    M = xc.size // 3

    table = voxel.reshape(C, D * H * W).T

    sizes = jnp.array([W - 1, H - 1, D - 1], jnp.float32)
    r_mult = jnp.where(jnp.arange(3) == ratio_dim, ratio, 1.0)
    a = 0.5 * sizes * r_mult / scale
    b = 0.5 * sizes * (1.0 - r_mult * offset / scale)
    par = jnp.concatenate(
        [jnp.stack([a[0], b[0], a[1], b[1], a[2], b[2]]),
         jnp.zeros((2,), jnp.float32)])
    par16 = jnp.broadcast_to(par[:, None], (8, L))

    out = _build_sc_call(M, C, D, H, W)(xc.reshape(-1), table, par16)
    return out.reshape(shape[:-1] + (C,))


# speedup vs baseline: 1.2324x; 1.0284x over previous
import functools

import jax
import jax.numpy as jnp
from jax import lax
from jax.experimental import pallas as pl
from jax.experimental.pallas import tpu as pltpu
from jax.experimental.pallas import tpu_sc as plsc

L = 16


def _iota():
    return lax.broadcasted_iota(jnp.int32, (L,), 0)


@functools.cache
def _build_sc_call(M, C, D, H, W):
    info = plsc.get_sparse_core_info()
    NC, NS = info.num_cores, info.num_subcores
    NW = NC * NS
    PW = M // NW
    B = 128
    G = B // L
    NCHUNK = PW // B
    assert M % NW == 0 and PW % B == 0

    mesh = plsc.VectorSubcoreMesh(core_axis_name="c", subcore_axis_name="s")

    @functools.partial(
        pl.kernel,
        out_type=jax.ShapeDtypeStruct((M, C), jnp.float32),
        mesh=mesh,
        compiler_params=pltpu.CompilerParams(needs_layout_passes=False,
                                             use_tc_tiling_on_sc=False),
        scratch_types=[
            pltpu.VMEM((B * 3,), jnp.float32),
            pltpu.VMEM((G, 8 * L), jnp.int32),
            pltpu.VMEM((G * 8 * L, C), jnp.float32),
            pltpu.VMEM((G, 8, L), jnp.float32),
            pltpu.VMEM((B, C), jnp.float32),
            pltpu.VMEM((8, L), jnp.float32),
            pltpu.SemaphoreType.DMA,
        ],
    )
    def grid_sample_sc(xc_hbm, table_hbm, par_hbm, out_hbm,
                       coords_v, idx_v, rows_v, w_v, out_v, par_v, sem):
        wid = lax.axis_index("s") * NC + lax.axis_index("c")
        wbase = wid * PW
        pltpu.sync_copy(par_hbm, par_v)
        lane = _iota()

        def chunk_body(ci, carry):
            base = wbase + ci * B
            pltpu.sync_copy(xc_hbm.at[pl.ds(base * 3, B * 3)], coords_v)

            for g in range(G):
                a3 = (g * L + lane) * 3
                gx = plsc.load_gather(coords_v, [a3])
                gy = plsc.load_gather(coords_v, [a3 + 1])
                gz = plsc.load_gather(coords_v, [a3 + 2])

                def axis(gc, arow, brow, size):
                    A = par_v[arow]
                    Bc = par_v[brow]
                    i = jnp.minimum(jnp.maximum(A * gc + Bc, 0.0),
                                    float(size - 1))
                    i0 = i.astype(jnp.int32)
                    w1 = i - i0.astype(jnp.float32)
                    i1 = jnp.minimum(i0 + 1, size - 1)
                    return i0, i1, w1

                x0, x1, wx = axis(gx, 0, 1, W)
                y0, y1, wy = axis(gy, 2, 3, H)
                z0, z1, wz = axis(gz, 4, 5, D)

                ux0, ux1 = 1.0 - wx, wx
                uy0, uy1 = 1.0 - wy, wy
                uz0, uz1 = 1.0 - wz, wz
                zy = [z0 * (H * W) + y0 * W, z0 * (H * W) + y1 * W,
                      z1 * (H * W) + y0 * W, z1 * (H * W) + y1 * W]
                wzy = [uz0 * uy0, uz0 * uy1, uz1 * uy0, uz1 * uy1]
                for k in range(8):
                    dz_dy = (k >> 1)
                    dx = k & 1
                    r = zy[dz_dy] + (x0 if dx == 0 else x1)
                    wk = wzy[dz_dy] * (ux0 if dx == 0 else ux1)
                    idx_v[g, pl.ds(k * L, L)] = r
                    w_v[g, k] = wk

            descs = [
                pltpu.async_copy(table_hbm.at[idx_v.at[g]],
                                 rows_v.at[pl.ds(g * 8 * L, 8 * L)], sem)
                for g in range(G)
            ]
            for d in descs:
                d.wait()

            for g in range(G):
                rbase = [g * 8 * L + k * L + lane for k in range(8)]
                wk = [w_v[g, k] for k in range(8)]
                orow = g * L + lane
                for c in range(C):
                    col = jnp.full((L,), c, jnp.int32)
                    acc = plsc.load_gather(rows_v, [rbase[0], col]) * wk[0]
                    for k in range(1, 8):
                        acc = acc + plsc.load_gather(rows_v, [rbase[k], col]) * wk[k]
                    plsc.store_scatter(out_v, [orow, col], acc)

            pltpu.sync_copy(out_v, out_hbm.at[pl.ds(base, B)])
            return carry

        lax.fori_loop(0, NCHUNK, chunk_body, 0)

    return grid_sample_sc


def kernel(xc, voxel, offset, scale, ratio, ratio_dim):
    shape = xc.shape
    C, D, H, W = voxel.shape
    M = xc.size // 3

    table = voxel.reshape(C, D * H * W).T

    sizes = jnp.array([W - 1, H - 1, D - 1], jnp.float32)
    r_mult = jnp.where(jnp.arange(3) == ratio_dim, ratio, 1.0)
    a = 0.5 * sizes * r_mult / scale
    b = 0.5 * sizes * (1.0 - r_mult * offset / scale)
    par = jnp.concatenate(
        [jnp.stack([a[0], b[0], a[1], b[1], a[2], b[2]]),
         jnp.zeros((2,), jnp.float32)])
    par16 = jnp.broadcast_to(par[:, None], (8, L))

    out = _build_sc_call(M, C, D, H, W)(xc.reshape(-1), table, par16)
    return out.reshape(shape[:-1] + (C,))


# speedup vs baseline: 2.9487x; 2.3926x over previous
import functools

import jax
import jax.numpy as jnp
from jax import lax
from jax.experimental import pallas as pl
from jax.experimental.pallas import tpu as pltpu
from jax.experimental.pallas import tpu_sc as plsc

L = 16


def _iota():
    return lax.broadcasted_iota(jnp.int32, (L,), 0)


@functools.cache
def _build_sc_call(M, C, D, H, W):
    info = plsc.get_sparse_core_info()
    NC, NS = info.num_cores, info.num_subcores
    NW = NC * NS
    PW = M // NW
    NG = PW // L
    CH = C // L
    assert M % NW == 0 and PW % (2 * L) == 0 and C % L == 0

    mesh = plsc.VectorSubcoreMesh(core_axis_name="c", subcore_axis_name="s")

    @functools.partial(
        pl.kernel,
        out_type=jax.ShapeDtypeStruct((M, C), jnp.float32),
        mesh=mesh,
        compiler_params=pltpu.CompilerParams(needs_layout_passes=False,
                                             use_tc_tiling_on_sc=False),
        scratch_types=[
            pltpu.VMEM((PW * 3,), jnp.float32),
            pltpu.VMEM((2, 8 * L), jnp.int32),
            pltpu.VMEM((2, 8 * L, C), jnp.float32),
            pltpu.VMEM((2, 8, L), jnp.float32),
            pltpu.VMEM((2, L, C), jnp.float32),
            pltpu.VMEM((8, L), jnp.float32),
            pltpu.SemaphoreType.DMA,
            pltpu.SemaphoreType.DMA,
        ],
    )
    def grid_sample_sc(xc_hbm, table_hbm, par_hbm, out_hbm,
                       coords_v, idx_v, rows_v, w_v, out_v, par_v,
                       sem_g, sem_o):
        wid = lax.axis_index("s") * NC + lax.axis_index("c")
        wbase = wid * PW
        pltpu.sync_copy(par_hbm, par_v)
        pltpu.sync_copy(xc_hbm.at[pl.ds(wbase * 3, PW * 3)], coords_v)
        lane = _iota()

        def a_phase(g, par):
            a3 = (g * L + lane) * 3
            gx = plsc.load_gather(coords_v, [a3])
            gy = plsc.load_gather(coords_v, [a3 + 1])
            gz = plsc.load_gather(coords_v, [a3 + 2])

            def axis(gc, arow, brow, size):
                i = jnp.minimum(jnp.maximum(par_v[arow] * gc + par_v[brow],
                                            0.0), float(size - 1))
                i0 = i.astype(jnp.int32)
                w1 = i - i0.astype(jnp.float32)
                i1 = jnp.minimum(i0 + 1, size - 1)
                return i0, i1, w1

            x0, x1, wx = axis(gx, 0, 1, W)
            y0, y1, wy = axis(gy, 2, 3, H)
            z0, z1, wz = axis(gz, 4, 5, D)

            zy = [z0 * (H * W) + y0 * W, z0 * (H * W) + y1 * W,
                  z1 * (H * W) + y0 * W, z1 * (H * W) + y1 * W]
            wzy = [(1.0 - wz) * (1.0 - wy), (1.0 - wz) * wy,
                   wz * (1.0 - wy), wz * wy]
            for k in range(8):
                r = zy[k >> 1] + (x0 if k & 1 == 0 else x1)
                wk = wzy[k >> 1] * ((1.0 - wx) if k & 1 == 0 else wx)
                idx_v[par, pl.ds(k * L, L)] = r
                w_v[par, k] = wk

        def fire_gather(par):
            pltpu.async_copy(table_hbm.at[idx_v.at[par]], rows_v.at[par],
                             sem_g)

        def wait_gather(par):
            pltpu.make_async_copy(table_hbm.at[idx_v.at[par]],
                                  rows_v.at[par], sem_g).wait()

        def b_phase(g, par):
            wv = [w_v[par, k] for k in range(8)]
            for p in range(L):
                wb = [jnp.full((L,), wv[k][p], jnp.float32)
                      for k in range(8)]
                for h in range(CH):
                    cs = pl.ds(h * L, L)
                    t = [rows_v[par, k * L + p, cs] * wb[k] for k in range(8)]
                    s01, s23 = t[0] + t[1], t[2] + t[3]
                    s45, s67 = t[4] + t[5], t[6] + t[7]
                    out_v[par, p, cs] = (s01 + s23) + (s45 + s67)

        def fire_out(g, par):
            pltpu.async_copy(out_v.at[par],
                             out_hbm.at[pl.ds(wbase + g * L, L)], sem_o)

        def drain_out(g, par):
            pltpu.make_async_copy(out_v.at[par],
                                  out_hbm.at[pl.ds(wbase + g * L, L)],
                                  sem_o).wait()

        a_phase(0, 0)
        fire_gather(0)

        def body(i, carry):
            for par in (0, 1):
                g = i * 2 + par
                nxt = 1 - par

                @pl.when(g + 1 < NG)
                def _prefetch():
                    a_phase(g + 1, nxt)
                    fire_gather(nxt)

                wait_gather(par)

                @pl.when(g >= 2)
                def _drain():
                    drain_out(g - 2, par)

                b_phase(g, par)
                fire_out(g, par)
            return carry

        lax.fori_loop(0, NG // 2, body, 0)
        drain_out(NG - 2, 0)
        drain_out(NG - 1, 1)

    return grid_sample_sc


def kernel(xc, voxel, offset, scale, ratio, ratio_dim):
    shape = xc.shape
    C, D, H, W = voxel.shape
    M = xc.size // 3

    table = voxel.reshape(C, D * H * W).T

    sizes = jnp.array([W - 1, H - 1, D - 1], jnp.float32)
    r_mult = jnp.where(jnp.arange(3) == ratio_dim, ratio, 1.0)
    a = 0.5 * sizes * r_mult / scale
    b = 0.5 * sizes * (1.0 - r_mult * offset / scale)
    par = jnp.concatenate(
        [jnp.stack([a[0], b[0], a[1], b[1], a[2], b[2]]),
         jnp.zeros((2,), jnp.float32)])
    par16 = jnp.broadcast_to(par[:, None], (8, L))

    out = _build_sc_call(M, C, D, H, W)(xc.reshape(-1), table, par16)
    return out.reshape(shape[:-1] + (C,))


# speedup vs baseline: 3.4249x; 1.1615x over previous
import functools

import jax
import jax.numpy as jnp
from jax import lax
from jax.experimental import pallas as pl
from jax.experimental.pallas import tpu as pltpu
from jax.experimental.pallas import tpu_sc as plsc

L = 16


def _iota():
    return lax.broadcasted_iota(jnp.int32, (L,), 0)


@functools.cache
def _build_transpose(NCELL, C):
    info = plsc.get_sparse_core_info()
    NC, NS = info.num_cores, info.num_subcores
    NW = NC * NS
    CW = NCELL // NW
    BLK = 768
    NB = CW // BLK
    P = BLK + 1
    assert NCELL % NW == 0 and CW % (2 * BLK) == 0 and C == 2 * L

    mesh = plsc.VectorSubcoreMesh(core_axis_name="c", subcore_axis_name="s")

    @functools.partial(
        pl.kernel,
        out_type=jax.ShapeDtypeStruct((NCELL, C), jnp.float32),
        mesh=mesh,
        compiler_params=pltpu.CompilerParams(needs_layout_passes=False,
                                             use_tc_tiling_on_sc=False),
        scratch_types=[
            pltpu.VMEM((2, C, P), jnp.float32),
            pltpu.VMEM((2, BLK, C), jnp.float32),
            pltpu.SemaphoreType.DMA,
            pltpu.SemaphoreType.DMA,
        ],
    )
    def transpose_sc(vox_hbm, tab_hbm, stage_v, ob_v, sem_i, sem_o):
        wid = lax.axis_index("s") * NC + lax.axis_index("c")
        wbase = wid * CW
        lane = _iota()

        def fire_stage(b, par):
            pltpu.async_copy(vox_hbm.at[:, pl.ds(wbase + b * BLK, BLK)],
                             stage_v.at[par, :, pl.ds(0, BLK)], sem_i)

        def wait_stage(b, par):
            pltpu.make_async_copy(vox_hbm.at[:, pl.ds(wbase + b * BLK, BLK)],
                                  stage_v.at[par, :, pl.ds(0, BLK)],
                                  sem_i).wait()

        def fire_out(b, par):
            pltpu.async_copy(ob_v.at[par],
                             tab_hbm.at[pl.ds(wbase + b * BLK, BLK)], sem_o)

        def drain_out(b, par):
            pltpu.make_async_copy(ob_v.at[par],
                                  tab_hbm.at[pl.ds(wbase + b * BLK, BLK)],
                                  sem_o).wait()

        fire_stage(0, 0)
        a_lo = lane * P
        a_hi = (lane + L) * P

        def body(i, carry):
            for par in (0, 1):
                b = i * 2 + par
                nxt = 1 - par

                @pl.when(b + 1 < NB)
                def _prefetch():
                    fire_stage(b + 1, nxt)

                wait_stage(b, par)

                @pl.when(b >= 2)
                def _drain():
                    drain_out(b - 2, par)

                sp = stage_v.at[par]
                op = ob_v.at[par]

                def jbody(j4, carry2):
                    for u in range(4):
                        j = j4 * 4 + u
                        v0 = plsc.load_gather(sp, [lane, jnp.full((L,), j, jnp.int32)])
                        v1 = plsc.load_gather(sp, [lane + L, jnp.full((L,), j, jnp.int32)])
                        plsc.store_scatter(op, [jnp.full((L,), j, jnp.int32), lane], v0)
                        plsc.store_scatter(op, [jnp.full((L,), j, jnp.int32), lane + L], v1)
                    return carry2

                lax.fori_loop(0, BLK // 4, jbody, 0)
                fire_out(b, par)
            return carry

        lax.fori_loop(0, NB // 2, body, 0)
        drain_out(NB - 2, 0)
        drain_out(NB - 1, 1)

    return transpose_sc


@functools.cache
def _build_sc_call(M, C, D, H, W):
    info = plsc.get_sparse_core_info()
    NC, NS = info.num_cores, info.num_subcores
    NW = NC * NS
    PW = M // NW
    NG = PW // L
    CH = C // L
    assert M % NW == 0 and PW % (2 * L) == 0 and C % L == 0

    mesh = plsc.VectorSubcoreMesh(core_axis_name="c", subcore_axis_name="s")

    @functools.partial(
        pl.kernel,
        out_type=jax.ShapeDtypeStruct((M, C), jnp.float32),
        mesh=mesh,
        compiler_params=pltpu.CompilerParams(needs_layout_passes=False,
                                             use_tc_tiling_on_sc=False),
        scratch_types=[
            pltpu.VMEM((PW * 3,), jnp.float32),
            pltpu.VMEM((2, 8 * L), jnp.int32),
            pltpu.VMEM((2, 8 * L, C), jnp.float32),
            pltpu.VMEM((2, 8, L), jnp.float32),
            pltpu.VMEM((2, L, C), jnp.float32),
            pltpu.VMEM((8, L), jnp.float32),
            pltpu.SemaphoreType.DMA,
            pltpu.SemaphoreType.DMA,
        ],
    )
    def grid_sample_sc(xc_hbm, table_hbm, par_hbm, out_hbm,
                       coords_v, idx_v, rows_v, w_v, out_v, par_v,
                       sem_g, sem_o):
        wid = lax.axis_index("s") * NC + lax.axis_index("c")
        wbase = wid * PW
        pltpu.sync_copy(par_hbm, par_v)
        pltpu.sync_copy(xc_hbm.at[pl.ds(wbase * 3, PW * 3)], coords_v)
        lane = _iota()

        def a_phase(g, par):
            a3 = (g * L + lane) * 3
            gx = plsc.load_gather(coords_v, [a3])
            gy = plsc.load_gather(coords_v, [a3 + 1])
            gz = plsc.load_gather(coords_v, [a3 + 2])

            def axis(gc, arow, brow, size):
                i = jnp.minimum(jnp.maximum(par_v[arow] * gc + par_v[brow],
                                            0.0), float(size - 1))
                i0 = i.astype(jnp.int32)
                w1 = i - i0.astype(jnp.float32)
                i1 = jnp.minimum(i0 + 1, size - 1)
                return i0, i1, w1

            x0, x1, wx = axis(gx, 0, 1, W)
            y0, y1, wy = axis(gy, 2, 3, H)
            z0, z1, wz = axis(gz, 4, 5, D)

            zy = [z0 * (H * W) + y0 * W, z0 * (H * W) + y1 * W,
                  z1 * (H * W) + y0 * W, z1 * (H * W) + y1 * W]
            wzy = [(1.0 - wz) * (1.0 - wy), (1.0 - wz) * wy,
                   wz * (1.0 - wy), wz * wy]
            for k in range(8):
                r = zy[k >> 1] + (x0 if k & 1 == 0 else x1)
                wk = wzy[k >> 1] * ((1.0 - wx) if k & 1 == 0 else wx)
                idx_v[par, pl.ds(k * L, L)] = r
                w_v[par, k] = wk

        def fire_gather(par):
            pltpu.async_copy(table_hbm.at[idx_v.at[par]], rows_v.at[par],
                             sem_g)

        def wait_gather(par):
            pltpu.make_async_copy(table_hbm.at[idx_v.at[par]],
                                  rows_v.at[par], sem_g).wait()

        def b_phase(g, par):
            wv = [w_v[par, k] for k in range(8)]
            for p in range(L):
                wb = [jnp.full((L,), wv[k][p], jnp.float32)
                      for k in range(8)]
                for h in range(CH):
                    cs = pl.ds(h * L, L)
                    t = [rows_v[par, k * L + p, cs] * wb[k] for k in range(8)]
                    s01, s23 = t[0] + t[1], t[2] + t[3]
                    s45, s67 = t[4] + t[5], t[6] + t[7]
                    out_v[par, p, cs] = (s01 + s23) + (s45 + s67)

        def fire_out(g, par):
            pltpu.async_copy(out_v.at[par],
                             out_hbm.at[pl.ds(wbase + g * L, L)], sem_o)

        def drain_out(g, par):
            pltpu.make_async_copy(out_v.at[par],
                                  out_hbm.at[pl.ds(wbase + g * L, L)],
                                  sem_o).wait()

        a_phase(0, 0)
        fire_gather(0)

        def body(i, carry):
            for par in (0, 1):
                g = i * 2 + par
                nxt = 1 - par

                @pl.when(g + 1 < NG)
                def _prefetch():
                    a_phase(g + 1, nxt)
                    fire_gather(nxt)

                wait_gather(par)

                @pl.when(g >= 2)
                def _drain():
                    drain_out(g - 2, par)

                b_phase(g, par)
                fire_out(g, par)
            return carry

        lax.fori_loop(0, NG // 2, body, 0)
        drain_out(NG - 2, 0)
        drain_out(NG - 1, 1)

    return grid_sample_sc


def kernel(xc, voxel, offset, scale, ratio, ratio_dim):
    shape = xc.shape
    C, D, H, W = voxel.shape
    M = xc.size // 3

    table = _build_transpose(D * H * W, C)(voxel.reshape(C, D * H * W))

    sizes = jnp.array([W - 1, H - 1, D - 1], jnp.float32)
    r_mult = jnp.where(jnp.arange(3) == ratio_dim, ratio, 1.0)
    a = 0.5 * sizes * r_mult / scale
    b = 0.5 * sizes * (1.0 - r_mult * offset / scale)
    par = jnp.concatenate(
        [jnp.stack([a[0], b[0], a[1], b[1], a[2], b[2]]),
         jnp.zeros((2,), jnp.float32)])
    par16 = jnp.broadcast_to(par[:, None], (8, L))

    out = _build_sc_call(M, C, D, H, W)(xc.reshape(-1), table, par16)
    return out.reshape(shape[:-1] + (C,))
